# baseline (speedup 1.0000x reference)
import jax
import jax.numpy as jnp
from jax.experimental import pallas as pl
from jax.experimental.pallas import tpu as pltpu


def _support1_kernel(x_ref, w1_ref, s1_ref):
    s1_ref[...] = jnp.dot(x_ref[...], w1_ref[...],
                          preferred_element_type=jnp.float32)


def _layer1_kernel(adj_ref, s1_ref, b1_ref, w2_ref, hid_ref, s2_ref):
    acc = jnp.dot(adj_ref[...], s1_ref[...],
                  preferred_element_type=jnp.float32)
    h = jnp.maximum(acc + b1_ref[...], 0.0)
    hid_ref[...] = h
    s2_ref[...] = jnp.dot(h, w2_ref[...], preferred_element_type=jnp.float32)


def _layer2_kernel(adj_ref, s2_ref, out_ref):
    out_ref[...] = jnp.dot(adj_ref[...], s2_ref[...],
                           preferred_element_type=jnp.float32)


def kernel(x, adj, W1, b1, W2):
    n, nfeat = x.shape
    nhid = W1.shape[1]
    nclass = W2.shape[1]

    bm = 400
    bs = 2000

    s1 = pl.pallas_call(
        _support1_kernel,
        grid=(n // bs,),
        in_specs=[pl.BlockSpec((bs, nfeat), lambda i: (i, 0)),
                  pl.BlockSpec((nfeat, nhid), lambda i: (0, 0))],
        out_specs=pl.BlockSpec((bs, nhid), lambda i: (i, 0)),
        out_shape=jax.ShapeDtypeStruct((n, nhid), jnp.float32),
        compiler_params=pltpu.CompilerParams(
            dimension_semantics=("parallel",)),
    )(x, W1)

    hid, s2 = pl.pallas_call(
        _layer1_kernel,
        grid=(n // bm,),
        in_specs=[pl.BlockSpec((bm, n), lambda i: (i, 0)),
                  pl.BlockSpec((n, nhid), lambda i: (0, 0)),
                  pl.BlockSpec((1, nhid), lambda i: (0, 0)),
                  pl.BlockSpec((nhid, nclass), lambda i: (0, 0))],
        out_specs=[pl.BlockSpec((bm, nhid), lambda i: (i, 0)),
                   pl.BlockSpec((bm, nclass), lambda i: (i, 0))],
        out_shape=[jax.ShapeDtypeStruct((n, nhid), jnp.float32),
                   jax.ShapeDtypeStruct((n, nclass), jnp.float32)],
        compiler_params=pltpu.CompilerParams(
            dimension_semantics=("parallel",)),
    )(adj, s1, b1.reshape(1, nhid), W2)

    out = pl.pallas_call(
        _layer2_kernel,
        grid=(n // bm,),
        in_specs=[pl.BlockSpec((bm, n), lambda i: (i, 0)),
                  pl.BlockSpec((n, nclass), lambda i: (0, 0))],
        out_specs=pl.BlockSpec((bm, nclass), lambda i: (i, 0)),
        out_shape=jax.ShapeDtypeStruct((n, nclass), jnp.float32),
        compiler_params=pltpu.CompilerParams(
            dimension_semantics=("parallel",)),
    )(adj, s2)

    return (hid, out)
